# baseline (speedup 1.0000x reference)
import jax
import jax.numpy as jnp
from jax.experimental import pallas as pl
from jax.experimental.pallas import tpu as pltpu

_TILE_M = 1024


def _router_body(x_ref, wt_ref, o_ref):
    logits = jnp.dot(x_ref[...], wt_ref[...], preferred_element_type=jnp.float32)
    m = jnp.max(logits, axis=-1, keepdims=True)
    e = jnp.exp(logits - m)
    o_ref[...] = e / jnp.sum(e, axis=-1, keepdims=True)


def kernel(x, W, c):
    M, D = x.shape
    E = W.shape[0]
    wt = W.T
    probs = pl.pallas_call(
        _router_body,
        grid=(M // _TILE_M,),
        in_specs=[
            pl.BlockSpec((_TILE_M, D), lambda i: (i, 0)),
            pl.BlockSpec((D, E), lambda i: (0, 0)),
        ],
        out_specs=pl.BlockSpec((_TILE_M, E), lambda i: (i, 0)),
        out_shape=jax.ShapeDtypeStruct((M, E), jnp.float32),
        compiler_params=pltpu.CompilerParams(
            dimension_semantics=("arbitrary",),
            vmem_limit_bytes=120 * 1024 * 1024,
        ),
    )(x, wt)
    return probs


# speedup vs baseline: 1.2354x; 1.2354x over previous
import jax
import jax.numpy as jnp
from jax.experimental import pallas as pl
from jax.experimental.pallas import tpu as pltpu

_TILE_M = 4096


def _router_body(x_ref, wt_ref, o_ref):
    logits = jnp.dot(x_ref[...], wt_ref[...], preferred_element_type=jnp.float32)
    m = jnp.max(logits, axis=-1, keepdims=True)
    e = jnp.exp(logits - m)
    o_ref[...] = e / jnp.sum(e, axis=-1, keepdims=True)


def kernel(x, W, c):
    M, D = x.shape
    E = W.shape[0]
    wt = W.T
    probs = pl.pallas_call(
        _router_body,
        grid=(M // _TILE_M,),
        in_specs=[
            pl.BlockSpec((_TILE_M, D), lambda i: (i, 0)),
            pl.BlockSpec((D, E), lambda i: (0, 0)),
        ],
        out_specs=pl.BlockSpec((_TILE_M, E), lambda i: (i, 0)),
        out_shape=jax.ShapeDtypeStruct((M, E), jnp.float32),
        compiler_params=pltpu.CompilerParams(
            dimension_semantics=("arbitrary",),
            vmem_limit_bytes=120 * 1024 * 1024,
        ),
    )(x, wt)
    return probs
